# baseline (speedup 1.0000x reference)
import functools

import jax
import jax.numpy as jnp
from jax import lax
from jax.experimental import pallas as pl
from jax.experimental.pallas import tpu as pltpu
from jax.experimental.pallas import tpu_sc as plsc

LATENT_DIM = 64
LANES = 16


def _make_mf_kernel(batch, num_workers, nc):
    b_per_w = batch // num_workers
    mesh = plsc.VectorSubcoreMesh(core_axis_name="c", subcore_axis_name="s")

    @functools.partial(
        pl.kernel,
        mesh=mesh,
        out_type=jax.ShapeDtypeStruct((batch,), jnp.float32),
        scratch_types=[
            pltpu.VMEM((b_per_w,), jnp.int32),
            pltpu.VMEM((b_per_w,), jnp.int32),
            pltpu.VMEM((b_per_w, LATENT_DIM), jnp.float32),
            pltpu.VMEM((b_per_w, LATENT_DIM), jnp.float32),
            pltpu.VMEM((b_per_w,), jnp.float32),
            pltpu.SemaphoreType.DMA,
            pltpu.SemaphoreType.DMA,
        ],
        compiler_params=pltpu.CompilerParams(
            needs_layout_passes=False, use_tc_tiling_on_sc=False),
    )
    def mf(users_hbm, items_hbm, utab_hbm, itab_hbm, out_hbm,
           uidx_v, iidx_v, urows_v, irows_v, out_v, sem_u, sem_i):
        wid = lax.axis_index("s") * nc + lax.axis_index("c")
        base = wid * b_per_w

        pltpu.sync_copy(users_hbm.at[pl.ds(base, b_per_w)], uidx_v)
        pltpu.sync_copy(items_hbm.at[pl.ds(base, b_per_w)], iidx_v)
        cu = pltpu.async_copy(utab_hbm.at[uidx_v], urows_v, sem_u)
        ci = pltpu.async_copy(itab_hbm.at[iidx_v], irows_v, sem_i)
        cu.wait()
        ci.wait()

        lane_ids = lax.iota(jnp.int32, LANES)

        def group_body(g, carry):
            rows = g * LANES + lane_ids
            acc = jnp.zeros((LANES,), jnp.float32)
            for d in range(LATENT_DIM):
                col = jnp.full((LANES,), d, jnp.int32)
                uv = plsc.load_gather(urows_v, [rows, col])
                iv = plsc.load_gather(irows_v, [rows, col])
                acc = acc + uv * iv
            out_v[pl.ds(g * LANES, LANES)] = 1.0 / (1.0 + jnp.exp(-acc))
            return carry

        lax.fori_loop(0, b_per_w // LANES, group_body, 0)
        pltpu.sync_copy(out_v, out_hbm.at[pl.ds(base, b_per_w)])

    return mf


def kernel(users, items, embedding_user, embedding_item):
    info = plsc.get_sparse_core_info()
    num_workers = info.num_cores * info.num_subcores
    mf = _make_mf_kernel(users.shape[0], num_workers, info.num_cores)
    return mf(users.astype(jnp.int32), items.astype(jnp.int32),
              embedding_user, embedding_item)
